# Initial kernel scaffold; baseline (speedup 1.0000x reference)
import jax
import jax.numpy as jnp
from jax.experimental import pallas as pl


def kernel(x, gate_W, fc_W, proj_W):
    raise NotImplementedError("write your pallas kernel here")


# speedup vs baseline: 1.2939x; 1.2939x over previous
import functools

import jax
import jax.numpy as jnp
from jax import lax
from jax.experimental import pallas as pl
from jax.experimental.pallas import tpu as pltpu
from jax.experimental.pallas import tpu_sc as plsc

NE = 8
TOPK = 2
D = 1024
H = 4096
T = 2048
A = T * TOPK
BM = 256
G = A // BM + NE
PAD = G * BM
BH = 1024
HB = H // BH
CH = 512
NCH = A // CH

_f32 = jnp.float32
_i32 = jnp.int32


def _router_body(x_ref, gw_ref, pos_ref, w0_ref, w1_ref, blk_ref, loss_ref):
    x = x_ref[...]
    gw = gw_ref[...]
    logits = lax.dot_general(x, gw, (((1,), (1,)), ((), ())),
                             preferred_element_type=_f32)
    m = jnp.max(logits, axis=1, keepdims=True)
    p = jnp.exp(logits - m)
    probs = p / jnp.sum(p, axis=1, keepdims=True)

    e_iota = lax.broadcasted_iota(_i32, (T, NE), 1).astype(_f32)
    m0 = jnp.max(probs, axis=1, keepdims=True)
    e0 = jnp.min(jnp.where(probs == m0, e_iota, float(NE)), axis=1, keepdims=True)
    oh0 = (e_iota == e0)
    probs_m = jnp.where(oh0, -1.0, probs)
    m1 = jnp.max(probs_m, axis=1, keepdims=True)
    e1 = jnp.min(jnp.where(probs_m == m1, e_iota, float(NE)), axis=1, keepdims=True)
    oh1 = (e_iota == e1)

    s = m0 + m1
    w0_ref[...] = m0 / s
    w1_ref[...] = m1 / s

    ohf0 = oh0.astype(_f32)
    ohf1 = oh1.astype(_f32)
    cnt8 = jnp.sum(ohf0 + ohf1, axis=0, keepdims=True)
    mean_probs = jnp.sum(probs, axis=0, keepdims=True) / float(T)
    loss = jnp.sum(mean_probs * (cnt8 / float(T)), axis=1, keepdims=True) * float(NE)
    loss_ref[...] = jnp.broadcast_to(loss, (8, 128))

    onehot = jnp.concatenate([ohf0, ohf1], axis=0)
    tri = (lax.broadcasted_iota(_i32, (CH, CH), 1)
           <= lax.broadcasted_iota(_i32, (CH, CH), 0)).astype(_f32)
    incs = []
    tots = []
    for c in range(NCH):
        oc = onehot[c * CH:(c + 1) * CH]
        cc = lax.dot_general(tri, oc, (((1,), (0,)), ((), ())),
                             preferred_element_type=_f32)
        incs.append(cc)
        tots.append(cc[CH - 1:CH, :])
    stot = jnp.concatenate(tots, axis=0)
    tre = (lax.broadcasted_iota(_i32, (NCH, NCH), 1)
           < lax.broadcasted_iota(_i32, (NCH, NCH), 0)).astype(_f32)
    off = lax.dot_general(tre, stot, (((1,), (0,)), ((), ())),
                          preferred_element_type=_f32)
    prank = jnp.concatenate(
        [incs[c] + off[c:c + 1, :] for c in range(NCH)], axis=0)
    rank = jnp.sum(prank * onehot, axis=1, keepdims=True) - 1.0

    cnt = jnp.sum(stot, axis=0, keepdims=True)
    padded = jnp.ceil(cnt / float(BM)) * float(BM)
    tru = (lax.broadcasted_iota(_i32, (NE, NE), 0)
           < lax.broadcasted_iota(_i32, (NE, NE), 1)).astype(_f32)
    start = lax.dot_general(padded, tru, (((1,), (0,)), ((), ())),
                            preferred_element_type=_f32)
    start_sel = jnp.sum(start * onehot, axis=1, keepdims=True)
    pos_ref[...] = (start_sel + rank).astype(_i32)

    gstart = lax.broadcasted_iota(_i32, (G, NE), 0).astype(_f32) * float(BM)
    blk = jnp.sum((gstart >= start).astype(_f32), axis=1, keepdims=True) - 1.0
    blk_ref[...] = blk.astype(_i32)


def _mm_body(blk_ref, x_ref, fc_ref, pj_ref, y_ref):
    @pl.when(pl.program_id(1) == 0)
    def _():
        y_ref[...] = jnp.zeros_like(y_ref)

    h = lax.dot_general(x_ref[...], fc_ref[0], (((1,), (1,)), ((), ())),
                        preferred_element_type=_f32)
    h = jnp.maximum(h, 0.0)
    h = h * h
    y_ref[...] += lax.dot_general(h, pj_ref[0], (((1,), (1,)), ((), ())),
                                  preferred_element_type=_f32)


_router = pl.pallas_call(
    _router_body,
    out_shape=(
        jax.ShapeDtypeStruct((A, 1), _i32),
        jax.ShapeDtypeStruct((T, 1), _f32),
        jax.ShapeDtypeStruct((T, 1), _f32),
        jax.ShapeDtypeStruct((G, 1), _i32),
        jax.ShapeDtypeStruct((8, 128), _f32),
    ),
)

_mm = pl.pallas_call(
    _mm_body,
    grid_spec=pltpu.PrefetchScalarGridSpec(
        num_scalar_prefetch=1,
        grid=(G, HB),
        in_specs=[
            pl.BlockSpec((BM, D), lambda g, h, blk: (g, 0)),
            pl.BlockSpec((1, BH, D), lambda g, h, blk: (blk[g], h, 0)),
            pl.BlockSpec((1, D, BH), lambda g, h, blk: (blk[g], 0, h)),
        ],
        out_specs=pl.BlockSpec((BM, D), lambda g, h, blk: (g, 0)),
    ),
    out_shape=jax.ShapeDtypeStruct((PAD, D), _f32),
    compiler_params=pltpu.CompilerParams(
        dimension_semantics=("arbitrary", "arbitrary")),
)

def _dispatch_body(x_hbm, pos_hbm, xs_hbm, idx_v, buf_v, sem):
    wid = lax.axis_index("s") * 2 + lax.axis_index("c")
    for s in range(2):
        a0 = wid * 128 + s * 64
        t0 = jnp.where(a0 >= T, a0 - T, a0)
        pltpu.sync_copy(pos_hbm.at[pl.ds(a0, 64)], idx_v)
        pltpu.sync_copy(x_hbm.at[pl.ds(t0, 64)], buf_v)
        pltpu.async_copy(buf_v, xs_hbm.at[idx_v], sem).wait()


def _gather2_body(y_hbm, pos_hbm, y0_hbm, y1_hbm,
                  p0_v, p1_v, b0_v, b1_v, sem0, sem1):
    wid = lax.axis_index("s") * 2 + lax.axis_index("c")
    for s in range(2):
        tb = wid * 64 + s * 32
        pltpu.sync_copy(pos_hbm.at[pl.ds(tb, 32)], p0_v)
        pltpu.sync_copy(pos_hbm.at[pl.ds(T + tb, 32)], p1_v)
        c0 = pltpu.async_copy(y_hbm.at[p0_v], b0_v, sem0)
        c1 = pltpu.async_copy(y_hbm.at[p1_v], b1_v, sem1)
        c0.wait()
        c1.wait()
        pltpu.sync_copy(b0_v, y0_hbm.at[pl.ds(tb, 32)])
        pltpu.sync_copy(b1_v, y1_hbm.at[pl.ds(tb, 32)])


def _wcombine_body(y0_ref, y1_ref, w0_ref, w1_ref, out_ref):
    out_ref[...] = w0_ref[...] * y0_ref[...] + w1_ref[...] * y1_ref[...]


@functools.cache
def _sc_kernels():
    mesh = plsc.VectorSubcoreMesh(core_axis_name="c", subcore_axis_name="s")
    dispatch = pl.kernel(
        _dispatch_body,
        out_type=jax.ShapeDtypeStruct((PAD, D), _f32),
        mesh=mesh,
        scratch_types=[
            pltpu.VMEM((64,), _i32),
            pltpu.VMEM((64, D), _f32),
            pltpu.SemaphoreType.DMA,
        ],
    )
    gather2 = pl.kernel(
        _gather2_body,
        out_type=(
            jax.ShapeDtypeStruct((T, D), _f32),
            jax.ShapeDtypeStruct((T, D), _f32),
        ),
        mesh=mesh,
        scratch_types=[
            pltpu.VMEM((32,), _i32),
            pltpu.VMEM((32,), _i32),
            pltpu.VMEM((32, D), _f32),
            pltpu.VMEM((32, D), _f32),
            pltpu.SemaphoreType.DMA,
            pltpu.SemaphoreType.DMA,
        ],
    )
    return dispatch, gather2


_TW = 512

_wcombine = pl.pallas_call(
    _wcombine_body,
    grid=(T // _TW,),
    in_specs=[
        pl.BlockSpec((_TW, D), lambda i: (i, 0)),
        pl.BlockSpec((_TW, D), lambda i: (i, 0)),
        pl.BlockSpec((_TW, 1), lambda i: (i, 0)),
        pl.BlockSpec((_TW, 1), lambda i: (i, 0)),
    ],
    out_specs=pl.BlockSpec((_TW, D), lambda i: (i, 0)),
    out_shape=jax.ShapeDtypeStruct((T, D), _f32),
)


def kernel(x, gate_W, fc_W, proj_W):
    _dispatch, _gather2 = _sc_kernels()
    xf = x.reshape(T, D)
    pos2, w02, w12, blk2, loss2 = _router(xf, gate_W)
    pos = pos2.reshape(A)
    blk = blk2.reshape(G)
    x_sorted = _dispatch(xf, pos)
    y = _mm(blk, x_sorted, fc_W, proj_W)
    y0, y1 = _gather2(y, pos)
    out = _wcombine(y0, y1, w02, w12)
    return out.reshape(1, T, D), loss2[0, 0]


# speedup vs baseline: 1.4272x; 1.1030x over previous
import functools

import jax
import jax.numpy as jnp
from jax import lax
from jax.experimental import pallas as pl
from jax.experimental.pallas import tpu as pltpu
from jax.experimental.pallas import tpu_sc as plsc

NE = 8
TOPK = 2
D = 1024
H = 4096
T = 2048
A = T * TOPK
BM = 256
G = A // BM + NE
PAD = G * BM
BH = 1024
HB = H // BH
CH = 512
NCH = A // CH

_f32 = jnp.float32
_i32 = jnp.int32


def _router_body(x_ref, gw_ref, pos_ref, w16_ref, blk_ref, loss_ref):
    x = x_ref[...]
    gw = gw_ref[...]
    logits = lax.dot_general(x, gw, (((1,), (1,)), ((), ())),
                             preferred_element_type=_f32)
    m = jnp.max(logits, axis=1, keepdims=True)
    p = jnp.exp(logits - m)
    probs = p / jnp.sum(p, axis=1, keepdims=True)

    e_iota = lax.broadcasted_iota(_i32, (T, NE), 1).astype(_f32)
    m0 = jnp.max(probs, axis=1, keepdims=True)
    e0 = jnp.min(jnp.where(probs == m0, e_iota, float(NE)), axis=1, keepdims=True)
    oh0 = (e_iota == e0)
    probs_m = jnp.where(oh0, -1.0, probs)
    m1 = jnp.max(probs_m, axis=1, keepdims=True)
    e1 = jnp.min(jnp.where(probs_m == m1, e_iota, float(NE)), axis=1, keepdims=True)
    oh1 = (e_iota == e1)

    s = m0 + m1
    w16_ref[...] = jnp.broadcast_to(
        jnp.concatenate([m0 / s, m1 / s], axis=0), (A, 128))

    ohf0 = oh0.astype(_f32)
    ohf1 = oh1.astype(_f32)
    cnt8 = jnp.sum(ohf0 + ohf1, axis=0, keepdims=True)
    mean_probs = jnp.sum(probs, axis=0, keepdims=True) / float(T)
    loss = jnp.sum(mean_probs * (cnt8 / float(T)), axis=1, keepdims=True) * float(NE)
    loss_ref[...] = jnp.broadcast_to(loss, (8, 128))

    onehot = jnp.concatenate([ohf0, ohf1], axis=0)
    tri = (lax.broadcasted_iota(_i32, (CH, CH), 1)
           <= lax.broadcasted_iota(_i32, (CH, CH), 0)).astype(_f32)
    incs = []
    tots = []
    for c in range(NCH):
        oc = onehot[c * CH:(c + 1) * CH]
        cc = lax.dot_general(tri, oc, (((1,), (0,)), ((), ())),
                             preferred_element_type=_f32)
        incs.append(cc)
        tots.append(cc[CH - 1:CH, :])
    stot = jnp.concatenate(tots, axis=0)
    tre = (lax.broadcasted_iota(_i32, (NCH, NCH), 1)
           < lax.broadcasted_iota(_i32, (NCH, NCH), 0)).astype(_f32)
    off = lax.dot_general(tre, stot, (((1,), (0,)), ((), ())),
                          preferred_element_type=_f32)
    prank = jnp.concatenate(
        [incs[c] + off[c:c + 1, :] for c in range(NCH)], axis=0)
    rank = jnp.sum(prank * onehot, axis=1, keepdims=True) - 1.0

    cnt = jnp.sum(stot, axis=0, keepdims=True)
    padded = jnp.ceil(cnt / float(BM)) * float(BM)
    tru = (lax.broadcasted_iota(_i32, (NE, NE), 0)
           < lax.broadcasted_iota(_i32, (NE, NE), 1)).astype(_f32)
    start = lax.dot_general(padded, tru, (((1,), (0,)), ((), ())),
                            preferred_element_type=_f32)
    start_sel = jnp.sum(start * onehot, axis=1, keepdims=True)
    pos_ref[...] = (start_sel + rank).astype(_i32)

    gstart = lax.broadcasted_iota(_i32, (G, NE), 0).astype(_f32) * float(BM)
    blk = jnp.sum((gstart >= start).astype(_f32), axis=1, keepdims=True) - 1.0
    blk_ref[...] = blk.astype(_i32)


def _mm_body(blk_ref, x_ref, fc_ref, pj_ref, w_ref, y_ref, acc_ref):
    h_id = pl.program_id(0)
    g_id = pl.program_id(1)
    sl = pl.ds(g_id * BM, BM)

    h = lax.dot_general(x_ref[...], fc_ref[0], (((1,), (1,)), ((), ())),
                        preferred_element_type=_f32)
    h = jnp.maximum(h, 0.0)
    h = h * h
    contrib = lax.dot_general(h, pj_ref[0], (((1,), (1,)), ((), ())),
                              preferred_element_type=_f32)

    @pl.when(h_id == 0)
    def _():
        acc_ref[sl, :] = contrib

    @pl.when(h_id > 0)
    def _():
        acc_ref[sl, :] = acc_ref[sl, :] + contrib

    @pl.when(h_id == HB - 1)
    def _():
        y_ref[...] = acc_ref[sl, :] * w_ref[:, 0:1]


_router = pl.pallas_call(
    _router_body,
    out_shape=(
        jax.ShapeDtypeStruct((A, 1), _i32),
        jax.ShapeDtypeStruct((A, 128), _f32),
        jax.ShapeDtypeStruct((G, 1), _i32),
        jax.ShapeDtypeStruct((8, 128), _f32),
    ),
)

_mm = pl.pallas_call(
    _mm_body,
    grid_spec=pltpu.PrefetchScalarGridSpec(
        num_scalar_prefetch=1,
        grid=(HB, G),
        in_specs=[
            pl.BlockSpec((BM, D), lambda h, g, blk: (g, 0)),
            pl.BlockSpec((1, BH, D), lambda h, g, blk: (blk[g], h, 0)),
            pl.BlockSpec((1, D, BH), lambda h, g, blk: (blk[g], 0, h)),
            pl.BlockSpec((BM, 128), lambda h, g, blk: (g, 0)),
        ],
        out_specs=pl.BlockSpec(
            (BM, D), lambda h, g, blk: (jnp.where(h == HB - 1, g, G), 0)),
        scratch_shapes=[pltpu.VMEM((PAD, D), _f32)],
    ),
    out_shape=jax.ShapeDtypeStruct((PAD + BM, D), _f32),
    compiler_params=pltpu.CompilerParams(
        dimension_semantics=("arbitrary", "arbitrary")),
)

def _dispatch_body(x_hbm, pos_hbm, w16_hbm, xs_hbm, ws_hbm,
                   idx_v, buf_v, wbuf_v, sem, semw):
    wid = lax.axis_index("s") * 2 + lax.axis_index("c")
    for s in range(2):
        a0 = wid * 128 + s * 64
        t0 = jnp.where(a0 >= T, a0 - T, a0)
        pltpu.sync_copy(pos_hbm.at[pl.ds(a0, 64)], idx_v)
        pltpu.sync_copy(x_hbm.at[pl.ds(t0, 64)], buf_v)
        pltpu.sync_copy(w16_hbm.at[pl.ds(a0, 64)], wbuf_v)
        cx = pltpu.async_copy(buf_v, xs_hbm.at[idx_v], sem)
        cw = pltpu.async_copy(wbuf_v, ws_hbm.at[idx_v], semw)
        cx.wait()
        cw.wait()


def _gather2_body(y_hbm, pos_hbm, out_hbm,
                  p0_v, p1_v, b0_v, b1_v, sem0, sem1):
    wid = lax.axis_index("s") * 2 + lax.axis_index("c")
    for s in range(2):
        tb = wid * 64 + s * 32
        pltpu.sync_copy(pos_hbm.at[pl.ds(tb, 32)], p0_v)
        pltpu.sync_copy(pos_hbm.at[pl.ds(T + tb, 32)], p1_v)
        c0 = pltpu.async_copy(y_hbm.at[p0_v], b0_v, sem0)
        c1 = pltpu.async_copy(y_hbm.at[p1_v], b1_v, sem1)
        c0.wait()
        c1.wait()

        def tok(i, carry):
            for j in range(D // 16):
                sl = pl.ds(j * 16, 16)
                b0_v[i, sl] = b0_v[i, sl] + b1_v[i, sl]
            return carry

        lax.fori_loop(0, 32, tok, 0)
        pltpu.sync_copy(b0_v, out_hbm.at[pl.ds(tb, 32)])


@functools.cache
def _sc_kernels():
    mesh = plsc.VectorSubcoreMesh(core_axis_name="c", subcore_axis_name="s")
    dispatch = pl.kernel(
        _dispatch_body,
        out_type=(
            jax.ShapeDtypeStruct((PAD, D), _f32),
            jax.ShapeDtypeStruct((PAD, 128), _f32),
        ),
        mesh=mesh,
        scratch_types=[
            pltpu.VMEM((64,), _i32),
            pltpu.VMEM((64, D), _f32),
            pltpu.VMEM((64, 128), _f32),
            pltpu.SemaphoreType.DMA,
            pltpu.SemaphoreType.DMA,
        ],
    )
    gather2 = pl.kernel(
        _gather2_body,
        out_type=jax.ShapeDtypeStruct((T, D), _f32),
        mesh=mesh,
        scratch_types=[
            pltpu.VMEM((32,), _i32),
            pltpu.VMEM((32,), _i32),
            pltpu.VMEM((32, D), _f32),
            pltpu.VMEM((32, D), _f32),
            pltpu.SemaphoreType.DMA,
            pltpu.SemaphoreType.DMA,
        ],
    )
    return dispatch, gather2


def kernel(x, gate_W, fc_W, proj_W):
    _dispatch, _gather2 = _sc_kernels()
    xf = x.reshape(T, D)
    pos2, w16, blk2, loss2 = _router(xf, gate_W)
    pos = pos2.reshape(A)
    blk = blk2.reshape(G)
    x_sorted, w_sorted = _dispatch(xf, pos, w16)
    y = _mm(blk, x_sorted, fc_W, proj_W, w_sorted)
    out = _gather2(y, pos)
    return out.reshape(1, T, D), loss2[0, 0]
